# baseline (speedup 1.0000x reference)
import functools

import jax
import jax.numpy as jnp
from jax import lax
from jax.experimental import pallas as pl
from jax.experimental.pallas import tpu as pltpu
from jax.experimental.pallas import tpu_sc as plsc

_EPS = 1e-12
_NC = 2
_NS = 16
_NW = _NC * _NS
_CH = 128
_NB = 10
_Q = 256000
_VC2 = 6400


def _tc_build_paired(table_t):
    h, v = table_t.shape
    n_blk = _Q // _VC2
    last_blk = pl.cdiv(v, _VC2) - 1
    hw = h // 2

    def body(x1_ref, x2_ref, x3_ref, x4_ref, o_ref):
        r = lax.broadcasted_iota(jnp.int32, (h, hw), 0)
        c = lax.broadcasted_iota(jnp.int32, (h, hw), 1)
        ep_e = (r == 2 * c).astype(jnp.float32)
        ep_o = (r == 2 * c + 1).astype(jnp.float32)

        for q, x_ref in enumerate((x1_ref, x2_ref, x3_ref, x4_ref)):
            x = x_ref[...]
            tl = lax.bitcast_convert_type(lax.dot_general(
                x, ep_e, (((0,), (0,)), ((), ())),
                preferred_element_type=jnp.float32), jnp.int32)
            th = lax.bitcast_convert_type(lax.dot_general(
                x, ep_o, (((0,), (0,)), ((), ())),
                preferred_element_type=jnp.float32), jnp.int32)
            o_ref[:, q * hw:(q + 1) * hw] = (
                lax.shift_right_logical(tl, 16) | (th & jnp.int32(-65536)))

    return pl.pallas_call(
        body,
        grid=(n_blk,),
        in_specs=[
            pl.BlockSpec((h, _VC2), lambda i: (0, i)),
            pl.BlockSpec((h, _VC2), lambda i: (0, jnp.minimum(i + n_blk, last_blk))),
            pl.BlockSpec((h, _VC2), lambda i: (0, jnp.minimum(i + 2 * n_blk, last_blk))),
            pl.BlockSpec((h, _VC2), lambda i: (0, jnp.minimum(i + 3 * n_blk, last_blk))),
        ],
        out_specs=pl.BlockSpec((_VC2, 4 * hw), lambda i: (i, 0)),
        out_shape=jax.ShapeDtypeStruct((_Q, 4 * hw), jnp.int32),
        compiler_params=pltpu.CompilerParams(vmem_limit_bytes=100 * 1024 * 1024),
    )(table_t, table_t, table_t, table_t)


def _sc_gather(table, idx3, nb=_NB):
    nw, n_ch, ch = idx3.shape
    _, hw = table.shape
    n = nw * n_ch * ch
    _NB = nb
    assert n_ch % _NB == 0 and n_ch // _NB >= 2
    mesh = plsc.VectorSubcoreMesh(core_axis_name="c", subcore_axis_name="s")

    @functools.partial(
        pl.kernel,
        mesh=mesh,
        compiler_params=pltpu.CompilerParams(use_tc_tiling_on_sc=False),
        out_type=jax.ShapeDtypeStruct((n, hw), jnp.int32),
        scratch_types=[
            pltpu.VMEM((n_ch, ch), jnp.int32),
            pltpu.VMEM((n_ch, ch), jnp.int32),
            pltpu.VMEM((_NB, ch, hw), jnp.int32),
            pltpu.SemaphoreType.DMA((_NB,)),
        ],
    )
    def k(table_hbm, idx_hbm, out_hbm, idx_v, pidx_v, rows_v, gsem):
        c = lax.axis_index("c")
        s = lax.axis_index("s")
        wid = s * _NC + c
        base = wid * (n_ch * ch)
        pltpu.sync_copy(idx_hbm.at[wid], idx_v)

        def to_paired(j):
            d = 4 * _Q - 1
            for kk in range(ch // 16):
                a = idx_v[j, pl.ds(kk * 16, 16)]
                a4 = a + a + a + a
                q = jnp.where(
                    a < _Q, a4,
                    jnp.where(a < 2 * _Q, a4 - d,
                              jnp.where(a < 3 * _Q, a4 - 2 * d, a4 - 3 * d)))
                pidx_v[j, pl.ds(kk * 16, 16)] = q

        for b in range(_NB):
            to_paired(b)
            pltpu.async_copy(table_hbm.at[pidx_v.at[b]], rows_v.at[b], gsem.at[b])

        def round_body(r, carry):
            j0 = r * _NB
            for b in range(_NB):
                pltpu.make_async_copy(
                    table_hbm.at[pidx_v.at[b]], rows_v.at[b], gsem.at[b]
                ).wait()
                pltpu.sync_copy(rows_v.at[b], out_hbm.at[pl.ds(base + (j0 + b) * ch, ch)])
                to_paired(j0 + b + _NB)
                pltpu.async_copy(
                    table_hbm.at[pidx_v.at[j0 + b + _NB]], rows_v.at[b], gsem.at[b]
                )
            return carry

        n_rounds = n_ch // _NB - 1
        lax.fori_loop(0, n_rounds, round_body, 0)

        j0 = n_rounds * _NB
        for b in range(_NB):
            pltpu.make_async_copy(
                table_hbm.at[pidx_v.at[b]], rows_v.at[b], gsem.at[b]
            ).wait()
            pltpu.sync_copy(rows_v.at[b], out_hbm.at[pl.ds(base + (j0 + b) * ch, ch)])

    return k(table, idx3)


def _tc_ln(xp, pos_p, tte_p, lnw_p, lnb_p):
    b, s, hw = xp.shape
    h = 2 * hw
    sb = 8

    def body(x_ref, pos_ref, tte_ref, w_ref, b_ref, o_ref):
        w = x_ref[...]
        xe = lax.bitcast_convert_type(lax.shift_left(w, 16), jnp.float32)
        xo = lax.bitcast_convert_type(
            lax.shift_left(lax.shift_right_logical(w, 16), 16), jnp.float32)
        bias = pos_ref[...] + tte_ref[0:1, :]
        xx = jnp.concatenate([xe, xo], axis=-1) + bias[None]
        mu = jnp.mean(xx, axis=-1, keepdims=True)
        xc = xx - mu
        var = jnp.mean(xc * xc, axis=-1, keepdims=True)
        y = xc * lax.rsqrt(var + _EPS) * w_ref[...] + b_ref[...]
        hh = lax.broadcasted_iota(jnp.int32, (h, h), 0)
        mm = lax.broadcasted_iota(jnp.int32, (h, h), 1)
        ep = (mm == (lax.shift_right_logical(hh, 1) + hw * (hh & 1)))
        ep = ep.astype(jnp.float32)
        for j in range(y.shape[1]):
            o_ref[j] = lax.dot_general(
                ep, y[:, j, :], (((1,), (1,)), ((), ())),
                preferred_element_type=jnp.float32,
            )

    return pl.pallas_call(
        body,
        grid=(s // sb,),
        in_specs=[
            pl.BlockSpec((b, sb, hw), lambda i: (0, i, 0)),
            pl.BlockSpec((sb, h), lambda i: (i, 0)),
            pl.BlockSpec(tte_p.shape, lambda i: (0, 0)),
            pl.BlockSpec((1, h), lambda i: (0, 0)),
            pl.BlockSpec((1, h), lambda i: (0, 0)),
        ],
        out_specs=pl.BlockSpec((sb, h, b), lambda i: (i, 0, 0)),
        out_shape=jax.ShapeDtypeStruct((s, h, b), jnp.float32),
        compiler_params=pltpu.CompilerParams(vmem_limit_bytes=100 * 1024 * 1024),
    )(xp, pos_p, tte_p, lnw_p, lnb_p)


def kernel(input_ids, word_embeddings, position_embeddings, token_type_embeddings, ln_weight, ln_bias):
    b, s = input_ids.shape
    v, h = word_embeddings.shape
    chs = 40
    n_chk = b * chs // (_NW * _CH)
    assert n_chk * _NW * _CH == b * chs and s % chs == 0

    def perm_h(a):
        return jnp.concatenate([a[..., 0::2], a[..., 1::2]], axis=-1)

    paired = _tc_build_paired(jnp.transpose(word_embeddings))
    pview = paired.reshape(4 * _Q, h // 2)
    tte_p = perm_h(token_type_embeddings)
    lnw_p = perm_h(ln_weight).reshape(1, h)
    lnb_p = perm_h(ln_bias).reshape(1, h)
    ids32 = input_ids.astype(jnp.int32)
    outs = []
    for k in range(s // chs):
        idx3k = ids32[:, k * chs:(k + 1) * chs].reshape(_NW, n_chk, _CH)
        gk = _sc_gather(pview, idx3k, nb=min(_NB, n_chk // 2))
        outs.append(_tc_ln(
            gk.reshape(b, chs, h // 2),
            perm_h(position_embeddings[k * chs:(k + 1) * chs]),
            tte_p, lnw_p, lnb_p,
        ))
    return jnp.transpose(jnp.concatenate(outs, axis=0), (2, 0, 1))


# speedup vs baseline: 1.1419x; 1.1419x over previous
import functools

import jax
import jax.numpy as jnp
from jax import lax
from jax.experimental import pallas as pl
from jax.experimental.pallas import tpu as pltpu
from jax.experimental.pallas import tpu_sc as plsc

_EPS = 1e-12
_NC = 2
_NS = 16
_NW = _NC * _NS
_CH = 128
_NB = 10
_Q = 256000
_VC2 = 6400


def _tc_build_paired(table_t):
    h, v = table_t.shape
    n_blk = _Q // _VC2
    last_blk = pl.cdiv(v, _VC2) - 1
    hw = h // 2

    def body(x1_ref, x2_ref, x3_ref, x4_ref, o_ref):
        r = lax.broadcasted_iota(jnp.int32, (h, hw), 0)
        c = lax.broadcasted_iota(jnp.int32, (h, hw), 1)
        ep_e = (r == 2 * c).astype(jnp.float32)
        ep_o = (r == 2 * c + 1).astype(jnp.float32)

        for q, x_ref in enumerate((x1_ref, x2_ref, x3_ref, x4_ref)):
            x = x_ref[...]
            tl = lax.bitcast_convert_type(lax.dot_general(
                x, ep_e, (((0,), (0,)), ((), ())),
                preferred_element_type=jnp.float32), jnp.int32)
            th = lax.bitcast_convert_type(lax.dot_general(
                x, ep_o, (((0,), (0,)), ((), ())),
                preferred_element_type=jnp.float32), jnp.int32)
            o_ref[:, q * hw:(q + 1) * hw] = (
                lax.shift_right_logical(tl, 16) | (th & jnp.int32(-65536)))

    return pl.pallas_call(
        body,
        grid=(n_blk,),
        in_specs=[
            pl.BlockSpec((h, _VC2), lambda i: (0, i)),
            pl.BlockSpec((h, _VC2), lambda i: (0, jnp.minimum(i + n_blk, last_blk))),
            pl.BlockSpec((h, _VC2), lambda i: (0, jnp.minimum(i + 2 * n_blk, last_blk))),
            pl.BlockSpec((h, _VC2), lambda i: (0, jnp.minimum(i + 3 * n_blk, last_blk))),
        ],
        out_specs=pl.BlockSpec((_VC2, 4 * hw), lambda i: (i, 0)),
        out_shape=jax.ShapeDtypeStruct((_Q, 4 * hw), jnp.int32),
        compiler_params=pltpu.CompilerParams(vmem_limit_bytes=100 * 1024 * 1024),
    )(table_t, table_t, table_t, table_t)


def _sc_gather(table, idx3):
    nw, n_ch, ch = idx3.shape
    _, hw = table.shape
    n = nw * n_ch * ch
    assert n_ch % _NB == 0 and n_ch // _NB >= 2
    mesh = plsc.VectorSubcoreMesh(core_axis_name="c", subcore_axis_name="s")

    @functools.partial(
        pl.kernel,
        mesh=mesh,
        compiler_params=pltpu.CompilerParams(use_tc_tiling_on_sc=False),
        out_type=jax.ShapeDtypeStruct((n, hw), jnp.int32),
        scratch_types=[
            pltpu.VMEM((n_ch, ch), jnp.int32),
            pltpu.VMEM((n_ch, ch), jnp.int32),
            pltpu.VMEM((_NB, ch, hw), jnp.int32),
            pltpu.SemaphoreType.DMA((_NB,)),
        ],
    )
    def k(table_hbm, idx_hbm, out_hbm, idx_v, pidx_v, rows_v, gsem):
        c = lax.axis_index("c")
        s = lax.axis_index("s")
        wid = s * _NC + c
        base = wid * (n_ch * ch)
        pltpu.sync_copy(idx_hbm.at[wid], idx_v)

        def to_paired(j):
            d = 4 * _Q - 1
            for kk in range(ch // 16):
                a = idx_v[j, pl.ds(kk * 16, 16)]
                a4 = a + a + a + a
                q = jnp.where(
                    a < _Q, a4,
                    jnp.where(a < 2 * _Q, a4 - d,
                              jnp.where(a < 3 * _Q, a4 - 2 * d, a4 - 3 * d)))
                pidx_v[j, pl.ds(kk * 16, 16)] = q

        for b in range(_NB):
            to_paired(b)
            pltpu.async_copy(table_hbm.at[pidx_v.at[b]], rows_v.at[b], gsem.at[b])

        def round_body(r, carry):
            j0 = r * _NB
            for b in range(_NB):
                pltpu.make_async_copy(
                    table_hbm.at[pidx_v.at[b]], rows_v.at[b], gsem.at[b]
                ).wait()
                pltpu.sync_copy(rows_v.at[b], out_hbm.at[pl.ds(base + (j0 + b) * ch, ch)])
                to_paired(j0 + b + _NB)
                pltpu.async_copy(
                    table_hbm.at[pidx_v.at[j0 + b + _NB]], rows_v.at[b], gsem.at[b]
                )
            return carry

        n_rounds = n_ch // _NB - 1
        lax.fori_loop(0, n_rounds, round_body, 0)

        j0 = n_rounds * _NB
        for b in range(_NB):
            pltpu.make_async_copy(
                table_hbm.at[pidx_v.at[b]], rows_v.at[b], gsem.at[b]
            ).wait()
            pltpu.sync_copy(rows_v.at[b], out_hbm.at[pl.ds(base + (j0 + b) * ch, ch)])

    return k(table, idx3)


def _tc_ln(xp, pos_p, tte_p, lnw_p, lnb_p):
    b, s, hw = xp.shape
    h = 2 * hw
    sb = 8

    def body(x_ref, pos_ref, tte_ref, w_ref, b_ref, o_ref):
        w = x_ref[...]
        xe = lax.bitcast_convert_type(lax.shift_left(w, 16), jnp.float32)
        xo = lax.bitcast_convert_type(
            lax.shift_left(lax.shift_right_logical(w, 16), 16), jnp.float32)
        bias = pos_ref[...] + tte_ref[0:1, :]
        xx = jnp.concatenate([xe, xo], axis=-1) + bias[None]
        mu = jnp.mean(xx, axis=-1, keepdims=True)
        xc = xx - mu
        var = jnp.mean(xc * xc, axis=-1, keepdims=True)
        y = xc * lax.rsqrt(var + _EPS) * w_ref[...] + b_ref[...]
        hh = lax.broadcasted_iota(jnp.int32, (h, h), 0)
        mm = lax.broadcasted_iota(jnp.int32, (h, h), 1)
        ep = (mm == (lax.shift_right_logical(hh, 1) + hw * (hh & 1)))
        ep = ep.astype(jnp.float32)
        for j in range(y.shape[1]):
            o_ref[j] = lax.dot_general(
                ep, y[:, j, :], (((1,), (1,)), ((), ())),
                preferred_element_type=jnp.float32,
            )

    return pl.pallas_call(
        body,
        grid=(s // sb,),
        in_specs=[
            pl.BlockSpec((b, sb, hw), lambda i: (0, i, 0)),
            pl.BlockSpec((sb, h), lambda i: (i, 0)),
            pl.BlockSpec(tte_p.shape, lambda i: (0, 0)),
            pl.BlockSpec((1, h), lambda i: (0, 0)),
            pl.BlockSpec((1, h), lambda i: (0, 0)),
        ],
        out_specs=pl.BlockSpec((sb, h, b), lambda i: (i, 0, 0)),
        out_shape=jax.ShapeDtypeStruct((s, h, b), jnp.float32),
        compiler_params=pltpu.CompilerParams(vmem_limit_bytes=100 * 1024 * 1024),
    )(xp, pos_p, tte_p, lnw_p, lnb_p)


def kernel(input_ids, word_embeddings, position_embeddings, token_type_embeddings, ln_weight, ln_bias):
    b, s = input_ids.shape
    v, h = word_embeddings.shape
    n = b * s
    per_w = n // _NW
    n_ch = per_w // _CH
    assert per_w * _NW == n and n_ch * _CH == per_w
    idx3 = input_ids.astype(jnp.int32).reshape(_NW, n_ch, _CH)
    paired = _tc_build_paired(jnp.transpose(word_embeddings))
    g = _sc_gather(paired.reshape(4 * _Q, h // 2), idx3)

    def perm_h(a):
        return jnp.concatenate([a[..., 0::2], a[..., 1::2]], axis=-1)

    out_shb = _tc_ln(
        g.reshape(b, s, h // 2),
        perm_h(position_embeddings[:s]),
        perm_h(token_type_embeddings),
        perm_h(ln_weight).reshape(1, h),
        perm_h(ln_bias).reshape(1, h),
    )
    return jnp.transpose(out_shb, (2, 0, 1))
